# baseline (speedup 1.0000x reference)
import jax
import jax.numpy as jnp
from jax.experimental import pallas as pl
from jax.experimental.pallas import tpu as pltpu

_CH = 800_000
_NCHUNK = 8


def _dma_relay(x_hbm, o_hbm, b0, b1, g0, g1, s0, s1):
    bufs = (b0, b1)
    gsems = (g0, g1)
    ssems = (s0, s1)
    gathers = [None] * _NCHUNK
    scatters = [None] * _NCHUNK
    gathers[0] = pltpu.make_async_copy(
        x_hbm.at[:, pl.ds(0, _CH)], bufs[0], gsems[0]
    )
    gathers[0].start()
    for i in range(_NCHUNK):
        b = i % 2
        nb = (i + 1) % 2
        if i + 1 < _NCHUNK:
            if i - 1 >= 0:
                scatters[i - 1].wait()
            gathers[i + 1] = pltpu.make_async_copy(
                x_hbm.at[:, pl.ds((i + 1) * _CH, _CH)], bufs[nb], gsems[nb]
            )
            gathers[i + 1].start()
        gathers[i].wait()
        scatters[i] = pltpu.make_async_copy(
            bufs[b], o_hbm.at[:, pl.ds(i * _CH, _CH)], ssems[b]
        )
        scatters[i].start()
    scatters[_NCHUNK - 2].wait()
    scatters[_NCHUNK - 1].wait()


def kernel(edge_index):
    E = edge_index.shape[1]
    out = pl.pallas_call(
        _dma_relay,
        in_specs=[pl.BlockSpec(memory_space=pl.ANY)],
        out_specs=pl.BlockSpec(memory_space=pl.ANY),
        out_shape=jax.ShapeDtypeStruct((2, E), edge_index.dtype),
        scratch_shapes=[
            pltpu.VMEM((2, _CH), jnp.int32),
            pltpu.VMEM((2, _CH), jnp.int32),
            pltpu.SemaphoreType.DMA,
            pltpu.SemaphoreType.DMA,
            pltpu.SemaphoreType.DMA,
            pltpu.SemaphoreType.DMA,
        ],
    )(edge_index)
    return out


# speedup vs baseline: 1.0525x; 1.0525x over previous
import jax
import jax.numpy as jnp
from jax.experimental import pallas as pl
from jax.experimental.pallas import tpu as pltpu

_CH = 400_000
_NCHUNK = 16
_NBUF = 8
_DEPTH = 4


def _dma_relay(x_hbm, o_hbm, *refs):
    bufs = refs[:_NBUF]
    gsems = refs[_NBUF:2 * _NBUF]
    ssems = refs[2 * _NBUF:3 * _NBUF]
    gathers = [None] * _NCHUNK
    scatters = [None] * _NCHUNK

    def start_gather(j):
        gathers[j] = pltpu.make_async_copy(
            x_hbm.at[:, pl.ds(j * _CH, _CH)], bufs[j % _NBUF], gsems[j % _NBUF]
        )
        gathers[j].start()

    for j in range(_DEPTH):
        start_gather(j)
    for i in range(_NCHUNK):
        j = i + _DEPTH
        if j < _NCHUNK:
            if j - _NBUF >= 0:
                scatters[j - _NBUF].wait()
            start_gather(j)
        gathers[i].wait()
        scatters[i] = pltpu.make_async_copy(
            bufs[i % _NBUF], o_hbm.at[:, pl.ds(i * _CH, _CH)], ssems[i % _NBUF]
        )
        scatters[i].start()
    for i in range(_NCHUNK - _NBUF, _NCHUNK):
        scatters[i].wait()


def kernel(edge_index):
    E = edge_index.shape[1]
    out = pl.pallas_call(
        _dma_relay,
        in_specs=[pl.BlockSpec(memory_space=pl.ANY)],
        out_specs=pl.BlockSpec(memory_space=pl.ANY),
        out_shape=jax.ShapeDtypeStruct((2, E), edge_index.dtype),
        scratch_shapes=(
            [pltpu.VMEM((2, _CH), jnp.int32)] * _NBUF
            + [pltpu.SemaphoreType.DMA] * (2 * _NBUF)
        ),
    )(edge_index)
    return out


# speedup vs baseline: 1.0734x; 1.0199x over previous
import jax
import jax.numpy as jnp
from jax.experimental import pallas as pl

_BC = 1_600_000


def _copy_block(x_ref, o_ref):
    o_ref[...] = x_ref[...]


def kernel(edge_index):
    E = edge_index.shape[1]
    out = pl.pallas_call(
        _copy_block,
        grid=(E // _BC,),
        in_specs=[pl.BlockSpec((2, _BC), lambda i: (0, i))],
        out_specs=pl.BlockSpec((2, _BC), lambda i: (0, i)),
        out_shape=jax.ShapeDtypeStruct((2, E), edge_index.dtype),
    )(edge_index)
    return out


# speedup vs baseline: 1.0765x; 1.0029x over previous
import jax
import jax.numpy as jnp
from jax.experimental import pallas as pl

_BC = 1_280_000


def _copy_block(x_ref, o_ref):
    o_ref[...] = x_ref[...]


def kernel(edge_index):
    E = edge_index.shape[1]
    out = pl.pallas_call(
        _copy_block,
        grid=(E // _BC,),
        in_specs=[pl.BlockSpec((2, _BC), lambda i: (0, i))],
        out_specs=pl.BlockSpec((2, _BC), lambda i: (0, i)),
        out_shape=jax.ShapeDtypeStruct((2, E), edge_index.dtype),
    )(edge_index)
    return out
